# Initial kernel scaffold; baseline (speedup 1.0000x reference)
import jax
import jax.numpy as jnp
from jax.experimental import pallas as pl


def kernel(x, edge_index, W1, b1, W2, b2, Wc, bc):
    raise NotImplementedError("write your pallas kernel here")


# speedup vs baseline: 19.4610x; 19.4610x over previous
import functools

import jax
import jax.numpy as jnp
from jax import lax
from jax.experimental import pallas as pl
from jax.experimental.pallas import tpu as pltpu
from jax.experimental.pallas import tpu_sc as plsc

_NC = 2
_NS = 16
_NW = _NC * _NS
_CHUNK = 200
_LANES = 16
_NPAD = 10240


def _zero_rows(buf, nrows, width):
    @pl.loop(0, nrows)
    def _(r):
        for j in range(width // _LANES):
            buf.at[pl.ds(r, 1), pl.ds(j * _LANES, _LANES)][...] = (
                jnp.zeros((1, _LANES), jnp.float32))


def _fill_spmem(zsrc, nzero, acc, base_row, nrows):
    full, rem = nrows // nzero, nrows % nzero
    for i in range(full):
        pltpu.sync_copy(zsrc.at[pl.ds(0, nzero)],
                        acc.at[pl.ds(base_row + i * nzero, nzero)])
    if rem:
        pltpu.sync_copy(zsrc.at[pl.ds(0, rem)],
                        acc.at[pl.ds(base_row + full * nzero, rem)])


def _sc_degree(dst, n):
    e = dst.shape[0]
    per_w = e // _NW
    n_chunks = per_w // _CHUNK
    rows_per_sub = _NPAD // _NS
    mesh = plsc.VectorSubcoreMesh(core_axis_name="c", subcore_axis_name="s")

    @functools.partial(
        pl.kernel,
        out_type=jax.ShapeDtypeStruct((_NC, _NPAD, _LANES), jnp.float32),
        mesh=mesh,
        scratch_types=[
            pltpu.VMEM((_CHUNK,), jnp.int32),
            pltpu.VMEM((_CHUNK, _LANES), jnp.float32),
            pltpu.VMEM((128, _LANES), jnp.float32),
            pltpu.VMEM_SHARED((_NPAD, _LANES), jnp.float32),
        ],
    )
    def k(dst_hbm, out_hbm, didx, ones, zbuf, acc):
        c = lax.axis_index("c")
        s = lax.axis_index("s")
        wid = c * _NS + s
        @pl.loop(0, _CHUNK)
        def _(r):
            ones.at[pl.ds(r, 1), pl.ds(0, _LANES)][...] = (
                jnp.ones((1, _LANES), jnp.float32))
        _zero_rows(zbuf, 128, _LANES)
        base_row = s * rows_per_sub
        _fill_spmem(zbuf, 128, acc, base_row, rows_per_sub)
        plsc.subcore_barrier()
        base_e = wid * per_w
        @pl.loop(0, n_chunks)
        def _(ci):
            pltpu.sync_copy(dst_hbm.at[pl.ds(base_e + ci * _CHUNK, _CHUNK)],
                            didx)
            pltpu.sync_copy(ones, acc.at[didx], add=True)
        plsc.subcore_barrier()
        pltpu.sync_copy(acc.at[pl.ds(base_row, rows_per_sub)],
                        out_hbm.at[c, pl.ds(base_row, rows_per_sub)])

    return k(dst)


def _sc_agg(hp, src, dst):
    n, d = hp.shape
    e = src.shape[0]
    per_w = e // _NW
    n_chunks = per_w // _CHUNK
    rows_per_sub = _NPAD // _NS
    mesh = plsc.VectorSubcoreMesh(core_axis_name="c", subcore_axis_name="s")

    @functools.partial(
        pl.kernel,
        out_type=jax.ShapeDtypeStruct((_NC, _NPAD, d), jnp.float32),
        mesh=mesh,
        scratch_types=[
            pltpu.VMEM((_CHUNK,), jnp.int32),
            pltpu.VMEM((_CHUNK,), jnp.int32),
            pltpu.VMEM((_CHUNK, d), jnp.float32),
            pltpu.VMEM_SHARED((_NPAD, d), jnp.float32),
        ],
    )
    def k(hp_hbm, src_hbm, dst_hbm, out_hbm, sidx, didx, rows, acc):
        c = lax.axis_index("c")
        s = lax.axis_index("s")
        wid = c * _NS + s
        _zero_rows(rows, 128, d)
        base_row = s * rows_per_sub
        _fill_spmem(rows, 128, acc, base_row, rows_per_sub)
        plsc.subcore_barrier()
        base_e = wid * per_w
        @pl.loop(0, n_chunks)
        def _(ci):
            off = base_e + ci * _CHUNK
            pltpu.sync_copy(src_hbm.at[pl.ds(off, _CHUNK)], sidx)
            pltpu.sync_copy(dst_hbm.at[pl.ds(off, _CHUNK)], didx)
            pltpu.sync_copy(hp_hbm.at[sidx], rows)
            pltpu.sync_copy(rows, acc.at[didx], add=True)
        plsc.subcore_barrier()
        pltpu.sync_copy(acc.at[pl.ds(base_row, rows_per_sub)],
                        out_hbm.at[c, pl.ds(base_row, rows_per_sub)])

    return k(hp, src, dst)


_BLK = 1000


def _tc_stage1(x, W1, degp):
    n, d = x.shape
    h = W1.shape[1]
    grid = (n // _BLK,)

    def body(x_ref, w_ref, degp_ref, hp_ref, dinv_ref):
        deg = 1.0 + degp_ref[0, :, 0:1] + degp_ref[1, :, 0:1]
        dinv = lax.rsqrt(deg)
        acc = jnp.dot(x_ref[...], w_ref[...],
                      preferred_element_type=jnp.float32)
        hp_ref[...] = acc * dinv
        dinv_ref[...] = dinv

    return pl.pallas_call(
        body,
        grid=grid,
        in_specs=[
            pl.BlockSpec((_BLK, d), lambda i: (i, 0)),
            pl.BlockSpec((d, h), lambda i: (0, 0)),
            pl.BlockSpec((_NC, _BLK, _LANES), lambda i: (0, i, 0)),
        ],
        out_specs=[
            pl.BlockSpec((_BLK, h), lambda i: (i, 0)),
            pl.BlockSpec((_BLK, 1), lambda i: (i, 0)),
        ],
        out_shape=[
            jax.ShapeDtypeStruct((n, h), jnp.float32),
            jax.ShapeDtypeStruct((n, 1), jnp.float32),
        ],
    )(x, W1, degp)


def _tc_stage2(aggp, hp1, dinv, b1, W2):
    n, h = hp1.shape
    grid = (n // _BLK,)

    def body(aggp_ref, hp_ref, dinv_ref, b_ref, w_ref, out1_ref, hp2_ref):
        agg = aggp_ref[0] + aggp_ref[1]
        o1 = jnp.maximum(
            dinv_ref[...] * (agg + hp_ref[...]) + b_ref[...], 0.0)
        out1_ref[...] = o1
        hp2_ref[...] = dinv_ref[...] * jnp.dot(
            o1, w_ref[...], preferred_element_type=jnp.float32)

    return pl.pallas_call(
        body,
        grid=grid,
        in_specs=[
            pl.BlockSpec((_NC, _BLK, h), lambda i: (0, i, 0)),
            pl.BlockSpec((_BLK, h), lambda i: (i, 0)),
            pl.BlockSpec((_BLK, 1), lambda i: (i, 0)),
            pl.BlockSpec((1, h), lambda i: (0, 0)),
            pl.BlockSpec((h, h), lambda i: (0, 0)),
        ],
        out_specs=[
            pl.BlockSpec((_BLK, h), lambda i: (i, 0)),
            pl.BlockSpec((_BLK, h), lambda i: (i, 0)),
        ],
        out_shape=[
            jax.ShapeDtypeStruct((n, h), jnp.float32),
            jax.ShapeDtypeStruct((n, h), jnp.float32),
        ],
    )(aggp, hp1, dinv, b1[None, :], W2)


def _tc_stage3(aggp, hp2, dinv, b2, out1, Wc, bc):
    n, h = hp2.shape
    o = Wc.shape[1]
    grid = (n // _BLK,)

    def body(aggp_ref, hp_ref, dinv_ref, b_ref, out1_ref, wc_ref, bc_ref,
             out_ref):
        agg = aggp_ref[0] + aggp_ref[1]
        o2 = jnp.maximum(
            dinv_ref[...] * (agg + hp_ref[...]) + b_ref[...], 0.0)
        out_ref[...] = jnp.dot(o2 + out1_ref[...], wc_ref[...],
                               preferred_element_type=jnp.float32) + bc_ref[...]

    return pl.pallas_call(
        body,
        grid=grid,
        in_specs=[
            pl.BlockSpec((_NC, _BLK, h), lambda i: (0, i, 0)),
            pl.BlockSpec((_BLK, h), lambda i: (i, 0)),
            pl.BlockSpec((_BLK, 1), lambda i: (i, 0)),
            pl.BlockSpec((1, h), lambda i: (0, 0)),
            pl.BlockSpec((_BLK, h), lambda i: (i, 0)),
            pl.BlockSpec((h, o), lambda i: (0, 0)),
            pl.BlockSpec((1, o), lambda i: (0, 0)),
        ],
        out_specs=pl.BlockSpec((_BLK, o), lambda i: (i, 0)),
        out_shape=jax.ShapeDtypeStruct((n, o), jnp.float32),
    )(aggp, hp2, dinv, b2[None, :], out1, Wc, bc[None, :])


def kernel(x, edge_index, W1, b1, W2, b2, Wc, bc):
    n = x.shape[0]
    src = edge_index[0]
    dst = edge_index[1]
    degp = _sc_degree(dst, n)
    hp1, dinv = _tc_stage1(x, W1, degp)
    aggp1 = _sc_agg(hp1, src, dst)
    out1, hp2 = _tc_stage2(aggp1, hp1, dinv, b1, W2)
    aggp2 = _sc_agg(hp2, src, dst)
    return _tc_stage3(aggp2, hp2, dinv, b2, out1, Wc, bc)


# speedup vs baseline: 25.0743x; 1.2884x over previous
import functools

import jax
import jax.numpy as jnp
from jax import lax
from jax.experimental import pallas as pl
from jax.experimental.pallas import tpu as pltpu
from jax.experimental.pallas import tpu_sc as plsc

_NC = 2
_NS = 16
_NW = _NC * _NS
_LANES = 16
_NPAD = 10112


def _zero_rows(buf, nrows, width):
    @pl.loop(0, nrows)
    def _(r):
        for j in range(width // _LANES):
            buf.at[pl.ds(r, 1), pl.ds(j * _LANES, _LANES)][...] = (
                jnp.zeros((1, _LANES), jnp.float32))


def _fill_spmem(zsrc, nzero, acc, base_row, nrows):
    full, rem = nrows // nzero, nrows % nzero
    for i in range(full):
        pltpu.sync_copy(zsrc.at[pl.ds(0, nzero)],
                        acc.at[pl.ds(base_row + i * nzero, nzero)])
    if rem:
        pltpu.sync_copy(zsrc.at[pl.ds(0, rem)],
                        acc.at[pl.ds(base_row + full * nzero, rem)])


def _sc_degree(dst):
    e = dst.shape[0]
    per_w = e // _NW
    kc = 200
    n_chunks = per_w // kc
    n_pairs = n_chunks // 2
    rows_per_sub = _NPAD // _NS
    mesh = plsc.VectorSubcoreMesh(core_axis_name="c", subcore_axis_name="s")

    @functools.partial(
        pl.kernel,
        out_type=jax.ShapeDtypeStruct((_NC, _NPAD, _LANES), jnp.float32),
        mesh=mesh,
        scratch_types=[
            pltpu.VMEM((kc,), jnp.int32),
            pltpu.VMEM((kc,), jnp.int32),
            pltpu.VMEM((kc, _LANES), jnp.float32),
            pltpu.VMEM((128, _LANES), jnp.float32),
            pltpu.VMEM_SHARED((_NPAD, _LANES), jnp.float32),
            pltpu.SemaphoreType.DMA,
            pltpu.SemaphoreType.DMA,
        ],
    )
    def k(dst_hbm, out_hbm, eb0, eb1, ones, zbuf, acc, s0, s1):
        c = lax.axis_index("c")
        s = lax.axis_index("s")
        wid = c * _NS + s
        @pl.loop(0, kc)
        def _(r):
            ones.at[pl.ds(r, 1), pl.ds(0, _LANES)][...] = (
                jnp.ones((1, _LANES), jnp.float32))
        _zero_rows(zbuf, 128, _LANES)
        base_row = s * rows_per_sub
        _fill_spmem(zbuf, 128, acc, base_row, rows_per_sub)
        plsc.subcore_barrier()
        base_e = wid * per_w

        def dst_slice(chunk):
            return dst_hbm.at[pl.ds(base_e + chunk * kc, kc)]

        pltpu.async_copy(dst_slice(0), eb0, s0)

        @pl.loop(0, n_pairs)
        def _(p):
            a = p * 2
            pltpu.async_copy(dst_slice(a + 1), eb1, s1)
            pltpu.make_async_copy(dst_slice(a), eb0, s0).wait()
            pltpu.sync_copy(ones, acc.at[eb0], add=True)
            @pl.when(p < n_pairs - 1)
            def _():
                pltpu.async_copy(dst_slice(a + 2), eb0, s0)
            pltpu.make_async_copy(dst_slice(a + 1), eb1, s1).wait()
            pltpu.sync_copy(ones, acc.at[eb1], add=True)

        plsc.subcore_barrier()
        pltpu.sync_copy(acc.at[pl.ds(base_row, rows_per_sub)],
                        out_hbm.at[c, pl.ds(base_row, rows_per_sub)])

    return k(dst)


_KA = 128


def _sc_agg(hp, src, dst):
    n, d = hp.shape
    e = src.shape[0]
    per_w = e // _NW
    n_full = per_w // _KA
    tail = per_w - n_full * _KA
    n_pairs = n_full // 2
    rows_per_sub = _NPAD // _NS
    mesh = plsc.VectorSubcoreMesh(core_axis_name="c", subcore_axis_name="s")

    @functools.partial(
        pl.kernel,
        out_type=jax.ShapeDtypeStruct((_NC, _NPAD, d), jnp.float32),
        mesh=mesh,
        scratch_types=[
            pltpu.VMEM((_KA,), jnp.int32),
            pltpu.VMEM((_KA,), jnp.int32),
            pltpu.VMEM((_KA,), jnp.int32),
            pltpu.VMEM((_KA,), jnp.int32),
            pltpu.VMEM((max(tail, 8),), jnp.int32),
            pltpu.VMEM((max(tail, 8),), jnp.int32),
            pltpu.VMEM((_KA, d), jnp.float32),
            pltpu.VMEM((_KA, d), jnp.float32),
            pltpu.VMEM_SHARED((_NPAD, d), jnp.float32),
            pltpu.SemaphoreType.DMA,
            pltpu.SemaphoreType.DMA,
        ],
    )
    def k(hp_hbm, src_hbm, dst_hbm, out_hbm, sb0, db0, sb1, db1, sbt, dbt,
          rows0, rows1, acc, g0, g1):
        c = lax.axis_index("c")
        s = lax.axis_index("s")
        wid = c * _NS + s
        _zero_rows(rows0, 128, d)
        base_row = s * rows_per_sub
        _fill_spmem(rows0, 128, acc, base_row, rows_per_sub)
        plsc.subcore_barrier()
        base_e = wid * per_w

        def load_idx(chunk, sb, db, size=_KA):
            off = base_e + chunk * _KA
            pltpu.sync_copy(src_hbm.at[pl.ds(off, size)], sb)
            pltpu.sync_copy(dst_hbm.at[pl.ds(off, size)], db)

        load_idx(0, sb0, db0)
        pltpu.async_copy(hp_hbm.at[sb0], rows0, g0)

        @pl.loop(0, n_pairs)
        def _(p):
            a = p * 2
            load_idx(a + 1, sb1, db1)
            pltpu.async_copy(hp_hbm.at[sb1], rows1, g1)
            pltpu.make_async_copy(hp_hbm.at[sb0], rows0, g0).wait()
            pltpu.sync_copy(rows0, acc.at[db0], add=True)
            @pl.when(p < n_pairs - 1)
            def _():
                load_idx(a + 2, sb0, db0)
                pltpu.async_copy(hp_hbm.at[sb0], rows0, g0)
            pltpu.make_async_copy(hp_hbm.at[sb1], rows1, g1).wait()
            pltpu.sync_copy(rows1, acc.at[db1], add=True)

        if tail:
            load_idx(n_full, sbt, dbt, tail)
            pltpu.sync_copy(hp_hbm.at[sbt], rows0.at[pl.ds(0, tail)])
            pltpu.sync_copy(rows0.at[pl.ds(0, tail)], acc.at[dbt],
                            add=True)

        plsc.subcore_barrier()
        pltpu.sync_copy(acc.at[pl.ds(base_row, rows_per_sub)],
                        out_hbm.at[c, pl.ds(base_row, rows_per_sub)])

    return k(hp, src, dst)


_BLK = 1000


def _tc_stage1(x, W1, degp):
    n, d = x.shape
    h = W1.shape[1]
    grid = (n // _BLK,)

    def body(x_ref, w_ref, degp_ref, hp_ref, dinv_ref):
        deg = 1.0 + degp_ref[0, :, 0:1] + degp_ref[1, :, 0:1]
        dinv = lax.rsqrt(deg)
        acc = jnp.dot(x_ref[...], w_ref[...],
                      preferred_element_type=jnp.float32)
        hp_ref[...] = acc * dinv
        dinv_ref[...] = dinv

    return pl.pallas_call(
        body,
        grid=grid,
        in_specs=[
            pl.BlockSpec((_BLK, d), lambda i: (i, 0)),
            pl.BlockSpec((d, h), lambda i: (0, 0)),
            pl.BlockSpec((_NC, _BLK, _LANES), lambda i: (0, i, 0)),
        ],
        out_specs=[
            pl.BlockSpec((_BLK, h), lambda i: (i, 0)),
            pl.BlockSpec((_BLK, 1), lambda i: (i, 0)),
        ],
        out_shape=[
            jax.ShapeDtypeStruct((n, h), jnp.float32),
            jax.ShapeDtypeStruct((n, 1), jnp.float32),
        ],
    )(x, W1, degp)


def _tc_stage2(aggp, hp1, dinv, b1, W2):
    n, h = hp1.shape
    grid = (n // _BLK,)

    def body(aggp_ref, hp_ref, dinv_ref, b_ref, w_ref, out1_ref, hp2_ref):
        agg = aggp_ref[0] + aggp_ref[1]
        o1 = jnp.maximum(
            dinv_ref[...] * (agg + hp_ref[...]) + b_ref[...], 0.0)
        out1_ref[...] = o1
        hp2_ref[...] = dinv_ref[...] * jnp.dot(
            o1, w_ref[...], preferred_element_type=jnp.float32)

    return pl.pallas_call(
        body,
        grid=grid,
        in_specs=[
            pl.BlockSpec((_NC, _BLK, h), lambda i: (0, i, 0)),
            pl.BlockSpec((_BLK, h), lambda i: (i, 0)),
            pl.BlockSpec((_BLK, 1), lambda i: (i, 0)),
            pl.BlockSpec((1, h), lambda i: (0, 0)),
            pl.BlockSpec((h, h), lambda i: (0, 0)),
        ],
        out_specs=[
            pl.BlockSpec((_BLK, h), lambda i: (i, 0)),
            pl.BlockSpec((_BLK, h), lambda i: (i, 0)),
        ],
        out_shape=[
            jax.ShapeDtypeStruct((n, h), jnp.float32),
            jax.ShapeDtypeStruct((n, h), jnp.float32),
        ],
    )(aggp, hp1, dinv, b1[None, :], W2)


def _tc_stage3(aggp, hp2, dinv, b2, out1, Wc, bc):
    n, h = hp2.shape
    o = Wc.shape[1]
    grid = (n // _BLK,)

    def body(aggp_ref, hp_ref, dinv_ref, b_ref, out1_ref, wc_ref, bc_ref,
             out_ref):
        agg = aggp_ref[0] + aggp_ref[1]
        o2 = jnp.maximum(
            dinv_ref[...] * (agg + hp_ref[...]) + b_ref[...], 0.0)
        out_ref[...] = jnp.dot(o2 + out1_ref[...], wc_ref[...],
                               preferred_element_type=jnp.float32) + bc_ref[...]

    return pl.pallas_call(
        body,
        grid=grid,
        in_specs=[
            pl.BlockSpec((_NC, _BLK, h), lambda i: (0, i, 0)),
            pl.BlockSpec((_BLK, h), lambda i: (i, 0)),
            pl.BlockSpec((_BLK, 1), lambda i: (i, 0)),
            pl.BlockSpec((1, h), lambda i: (0, 0)),
            pl.BlockSpec((_BLK, h), lambda i: (i, 0)),
            pl.BlockSpec((h, o), lambda i: (0, 0)),
            pl.BlockSpec((1, o), lambda i: (0, 0)),
        ],
        out_specs=pl.BlockSpec((_BLK, o), lambda i: (i, 0)),
        out_shape=jax.ShapeDtypeStruct((n, o), jnp.float32),
    )(aggp, hp2, dinv, b2[None, :], out1, Wc, bc[None, :])


def kernel(x, edge_index, W1, b1, W2, b2, Wc, bc):
    src = edge_index[0]
    dst = edge_index[1]
    degp = _sc_degree(dst)
    hp1, dinv = _tc_stage1(x, W1, degp)
    aggp1 = _sc_agg(hp1, src, dst)
    out1, hp2 = _tc_stage2(aggp1, hp1, dinv, b1, W2)
    aggp2 = _sc_agg(hp2, src, dst)
    return _tc_stage3(aggp2, hp2, dinv, b2, out1, Wc, bc)


# speedup vs baseline: 31.7428x; 1.2660x over previous
import functools

import jax
import jax.numpy as jnp
from jax import lax
from jax.experimental import pallas as pl
from jax.experimental.pallas import tpu as pltpu
from jax.experimental.pallas import tpu_sc as plsc

_NC = 2
_NS = 16
_NW = _NC * _NS
_LANES = 16
_NPAD = 10112


def _zero_rows(buf, nrows, width):
    @pl.loop(0, nrows)
    def _(r):
        for j in range(width // _LANES):
            buf.at[pl.ds(r, 1), pl.ds(j * _LANES, _LANES)][...] = (
                jnp.zeros((1, _LANES), jnp.float32))


def _fill_spmem(zsrc, nzero, acc, base_row, nrows):
    full, rem = nrows // nzero, nrows % nzero
    for i in range(full):
        pltpu.sync_copy(zsrc.at[pl.ds(0, nzero)],
                        acc.at[pl.ds(base_row + i * nzero, nzero)])
    if rem:
        pltpu.sync_copy(zsrc.at[pl.ds(0, rem)],
                        acc.at[pl.ds(base_row + full * nzero, rem)])


def _sc_degree(dst):
    e = dst.shape[0]
    per_w = e // _NW
    kc = 200
    n_chunks = per_w // kc
    n_pairs = n_chunks // 2
    rows_per_sub = _NPAD // _NS
    mesh = plsc.VectorSubcoreMesh(core_axis_name="c", subcore_axis_name="s")

    @functools.partial(
        pl.kernel,
        out_type=jax.ShapeDtypeStruct((_NC, _NPAD, _LANES), jnp.float32),
        mesh=mesh,
        scratch_types=[
            pltpu.VMEM((kc,), jnp.int32),
            pltpu.VMEM((kc,), jnp.int32),
            pltpu.VMEM((kc, _LANES), jnp.float32),
            pltpu.VMEM((128, _LANES), jnp.float32),
            pltpu.VMEM_SHARED((_NPAD, _LANES), jnp.float32),
            pltpu.SemaphoreType.DMA,
            pltpu.SemaphoreType.DMA,
        ],
    )
    def k(dst_hbm, out_hbm, eb0, eb1, ones, zbuf, acc, s0, s1):
        c = lax.axis_index("c")
        s = lax.axis_index("s")
        wid = c * _NS + s
        @pl.loop(0, kc)
        def _(r):
            ones.at[pl.ds(r, 1), pl.ds(0, _LANES)][...] = (
                jnp.ones((1, _LANES), jnp.float32))
        _zero_rows(zbuf, 128, _LANES)
        base_row = s * rows_per_sub
        _fill_spmem(zbuf, 128, acc, base_row, rows_per_sub)
        plsc.subcore_barrier()
        base_e = wid * per_w

        def dst_slice(chunk):
            return dst_hbm.at[pl.ds(base_e + chunk * kc, kc)]

        pltpu.async_copy(dst_slice(0), eb0, s0)

        @pl.loop(0, n_pairs)
        def _(p):
            a = p * 2
            pltpu.async_copy(dst_slice(a + 1), eb1, s1)
            pltpu.make_async_copy(dst_slice(a), eb0, s0).wait()
            pltpu.sync_copy(ones, acc.at[eb0], add=True)
            @pl.when(p < n_pairs - 1)
            def _():
                pltpu.async_copy(dst_slice(a + 2), eb0, s0)
            pltpu.make_async_copy(dst_slice(a + 1), eb1, s1).wait()
            pltpu.sync_copy(ones, acc.at[eb1], add=True)

        plsc.subcore_barrier()
        pltpu.sync_copy(acc.at[pl.ds(base_row, rows_per_sub)],
                        out_hbm.at[c, pl.ds(base_row, rows_per_sub)])

    return k(dst)


_KA = 128


def _sc_agg(hp, src, dst):
    n, d = hp.shape
    e = src.shape[0]
    per_w = e // _NW
    n_full = per_w // _KA
    tail = per_w - n_full * _KA
    n_pairs = n_full // 2
    rows_per_sub = _NPAD // _NS
    mesh = plsc.VectorSubcoreMesh(core_axis_name="c", subcore_axis_name="s")

    @functools.partial(
        pl.kernel,
        out_type=jax.ShapeDtypeStruct((_NC, _NPAD, d), jnp.float32),
        mesh=mesh,
        scratch_types=[
            pltpu.VMEM((_KA,), jnp.int32),
            pltpu.VMEM((_KA,), jnp.int32),
            pltpu.VMEM((_KA,), jnp.int32),
            pltpu.VMEM((_KA,), jnp.int32),
            pltpu.VMEM((max(tail, 8),), jnp.int32),
            pltpu.VMEM((max(tail, 8),), jnp.int32),
            pltpu.VMEM((_KA, d), jnp.float32),
            pltpu.VMEM((_KA, d), jnp.float32),
            pltpu.VMEM_SHARED((_NPAD, d), jnp.float32),
            pltpu.SemaphoreType.DMA,
            pltpu.SemaphoreType.DMA,
            pltpu.SemaphoreType.DMA,
            pltpu.SemaphoreType.DMA,
            pltpu.SemaphoreType.DMA,
            pltpu.SemaphoreType.DMA,
        ],
    )
    def k(hp_hbm, src_hbm, dst_hbm, out_hbm, sb0, db0, sb1, db1, sbt, dbt,
          rows0, rows1, acc, g0, g1, es0, es1, ed0, ed1):
        c = lax.axis_index("c")
        s = lax.axis_index("s")
        wid = c * _NS + s
        _zero_rows(rows0, 128, d)
        base_row = s * rows_per_sub
        _fill_spmem(rows0, 128, acc, base_row, rows_per_sub)
        plsc.subcore_barrier()
        base_e = wid * per_w

        def sidx(chunk, size=_KA):
            return src_hbm.at[pl.ds(base_e + chunk * _KA, size)]

        def didx(chunk, size=_KA):
            return dst_hbm.at[pl.ds(base_e + chunk * _KA, size)]

        pltpu.async_copy(sidx(0), sb0, es0)
        pltpu.async_copy(didx(0), db0, ed0)
        pltpu.async_copy(sidx(1), sb1, es1)
        pltpu.async_copy(didx(1), db1, ed1)
        pltpu.make_async_copy(sidx(0), sb0, es0).wait()
        pltpu.async_copy(hp_hbm.at[sb0], rows0, g0)
        pltpu.make_async_copy(sidx(1), sb1, es1).wait()
        pltpu.async_copy(hp_hbm.at[sb1], rows1, g1)

        @pl.loop(0, n_pairs)
        def _(p):
            a = p * 2
            more = p < n_pairs - 1
            pltpu.make_async_copy(hp_hbm.at[sb0], rows0, g0).wait()
            @pl.when(more)
            def _():
                pltpu.async_copy(sidx(a + 2), sb0, es0)
            pltpu.make_async_copy(didx(a), db0, ed0).wait()
            pltpu.sync_copy(rows0, acc.at[db0], add=True)
            @pl.when(more)
            def _():
                pltpu.async_copy(didx(a + 2), db0, ed0)
                pltpu.make_async_copy(sidx(a + 2), sb0, es0).wait()
                pltpu.async_copy(hp_hbm.at[sb0], rows0, g0)
            pltpu.make_async_copy(hp_hbm.at[sb1], rows1, g1).wait()
            @pl.when(more)
            def _():
                pltpu.async_copy(sidx(a + 3), sb1, es1)
            pltpu.make_async_copy(didx(a + 1), db1, ed1).wait()
            pltpu.sync_copy(rows1, acc.at[db1], add=True)
            @pl.when(more)
            def _():
                pltpu.async_copy(didx(a + 3), db1, ed1)
                pltpu.make_async_copy(sidx(a + 3), sb1, es1).wait()
                pltpu.async_copy(hp_hbm.at[sb1], rows1, g1)

        if tail:
            pltpu.sync_copy(sidx(n_full, tail), sbt)
            pltpu.sync_copy(didx(n_full, tail), dbt)
            pltpu.sync_copy(hp_hbm.at[sbt], rows0.at[pl.ds(0, tail)])
            pltpu.sync_copy(rows0.at[pl.ds(0, tail)], acc.at[dbt],
                            add=True)

        plsc.subcore_barrier()
        pltpu.sync_copy(acc.at[pl.ds(base_row, rows_per_sub)],
                        out_hbm.at[c, pl.ds(base_row, rows_per_sub)])

    return k(hp, src, dst)


_BLK = 1000


def _tc_stage1(x, W1, degp):
    n, d = x.shape
    h = W1.shape[1]
    grid = (n // _BLK,)

    def body(x_ref, w_ref, degp_ref, hp_ref, dinv_ref):
        deg = 1.0 + degp_ref[0, :, 0:1] + degp_ref[1, :, 0:1]
        dinv = lax.rsqrt(deg)
        acc = jnp.dot(x_ref[...], w_ref[...],
                      preferred_element_type=jnp.float32)
        hp_ref[...] = acc * dinv
        dinv_ref[...] = dinv

    return pl.pallas_call(
        body,
        grid=grid,
        in_specs=[
            pl.BlockSpec((_BLK, d), lambda i: (i, 0)),
            pl.BlockSpec((d, h), lambda i: (0, 0)),
            pl.BlockSpec((_NC, _BLK, _LANES), lambda i: (0, i, 0)),
        ],
        out_specs=[
            pl.BlockSpec((_BLK, h), lambda i: (i, 0)),
            pl.BlockSpec((_BLK, 1), lambda i: (i, 0)),
        ],
        out_shape=[
            jax.ShapeDtypeStruct((n, h), jnp.float32),
            jax.ShapeDtypeStruct((n, 1), jnp.float32),
        ],
    )(x, W1, degp)


def _tc_stage2(aggp, hp1, dinv, b1, W2):
    n, h = hp1.shape
    grid = (n // _BLK,)

    def body(aggp_ref, hp_ref, dinv_ref, b_ref, w_ref, out1_ref, hp2_ref):
        agg = aggp_ref[0] + aggp_ref[1]
        o1 = jnp.maximum(
            dinv_ref[...] * (agg + hp_ref[...]) + b_ref[...], 0.0)
        out1_ref[...] = o1
        hp2_ref[...] = dinv_ref[...] * jnp.dot(
            o1, w_ref[...], preferred_element_type=jnp.float32)

    return pl.pallas_call(
        body,
        grid=grid,
        in_specs=[
            pl.BlockSpec((_NC, _BLK, h), lambda i: (0, i, 0)),
            pl.BlockSpec((_BLK, h), lambda i: (i, 0)),
            pl.BlockSpec((_BLK, 1), lambda i: (i, 0)),
            pl.BlockSpec((1, h), lambda i: (0, 0)),
            pl.BlockSpec((h, h), lambda i: (0, 0)),
        ],
        out_specs=[
            pl.BlockSpec((_BLK, h), lambda i: (i, 0)),
            pl.BlockSpec((_BLK, h), lambda i: (i, 0)),
        ],
        out_shape=[
            jax.ShapeDtypeStruct((n, h), jnp.float32),
            jax.ShapeDtypeStruct((n, h), jnp.float32),
        ],
    )(aggp, hp1, dinv, b1[None, :], W2)


def _tc_stage3(aggp, hp2, dinv, b2, out1, Wc, bc):
    n, h = hp2.shape
    o = Wc.shape[1]
    grid = (n // _BLK,)

    def body(aggp_ref, hp_ref, dinv_ref, b_ref, out1_ref, wc_ref, bc_ref,
             out_ref):
        agg = aggp_ref[0] + aggp_ref[1]
        o2 = jnp.maximum(
            dinv_ref[...] * (agg + hp_ref[...]) + b_ref[...], 0.0)
        out_ref[...] = jnp.dot(o2 + out1_ref[...], wc_ref[...],
                               preferred_element_type=jnp.float32) + bc_ref[...]

    return pl.pallas_call(
        body,
        grid=grid,
        in_specs=[
            pl.BlockSpec((_NC, _BLK, h), lambda i: (0, i, 0)),
            pl.BlockSpec((_BLK, h), lambda i: (i, 0)),
            pl.BlockSpec((_BLK, 1), lambda i: (i, 0)),
            pl.BlockSpec((1, h), lambda i: (0, 0)),
            pl.BlockSpec((_BLK, h), lambda i: (i, 0)),
            pl.BlockSpec((h, o), lambda i: (0, 0)),
            pl.BlockSpec((1, o), lambda i: (0, 0)),
        ],
        out_specs=pl.BlockSpec((_BLK, o), lambda i: (i, 0)),
        out_shape=jax.ShapeDtypeStruct((n, o), jnp.float32),
    )(aggp, hp2, dinv, b2[None, :], out1, Wc, bc[None, :])


def kernel(x, edge_index, W1, b1, W2, b2, Wc, bc):
    src = edge_index[0]
    dst = edge_index[1]
    degp = _sc_degree(dst)
    hp1, dinv = _tc_stage1(x, W1, degp)
    aggp1 = _sc_agg(hp1, src, dst)
    out1, hp2 = _tc_stage2(aggp1, hp1, dinv, b1, W2)
    aggp2 = _sc_agg(hp2, src, dst)
    return _tc_stage3(aggp2, hp2, dinv, b2, out1, Wc, bc)


# speedup vs baseline: 33.8861x; 1.0675x over previous
import functools

import jax
import jax.numpy as jnp
from jax import lax
from jax.experimental import pallas as pl
from jax.experimental.pallas import tpu as pltpu
from jax.experimental.pallas import tpu_sc as plsc

_NC = 2
_NS = 16
_NW = _NC * _NS
_LANES = 16
_NPAD = 10112


def _zero_rows(buf, nrows, width):
    @pl.loop(0, nrows)
    def _(r):
        for j in range(width // _LANES):
            buf.at[pl.ds(r, 1), pl.ds(j * _LANES, _LANES)][...] = (
                jnp.zeros((1, _LANES), jnp.float32))


def _fill_spmem(zsrc, nzero, acc, base_row, nrows):
    full, rem = nrows // nzero, nrows % nzero
    for i in range(full):
        pltpu.sync_copy(zsrc.at[pl.ds(0, nzero)],
                        acc.at[pl.ds(base_row + i * nzero, nzero)])
    if rem:
        pltpu.sync_copy(zsrc.at[pl.ds(0, rem)],
                        acc.at[pl.ds(base_row + full * nzero, rem)])


def _sc_degree(dst):
    e = dst.shape[0]
    per_w = e // _NW
    kc = 200
    n_chunks = per_w // kc
    n_pairs = n_chunks // 2
    rows_per_sub = _NPAD // _NS
    mesh = plsc.VectorSubcoreMesh(core_axis_name="c", subcore_axis_name="s")

    @functools.partial(
        pl.kernel,
        out_type=jax.ShapeDtypeStruct((_NC, _NPAD, _LANES), jnp.float32),
        mesh=mesh,
        scratch_types=[
            pltpu.VMEM((kc,), jnp.int32),
            pltpu.VMEM((kc,), jnp.int32),
            pltpu.VMEM((kc, _LANES), jnp.float32),
            pltpu.VMEM((128, _LANES), jnp.float32),
            pltpu.VMEM_SHARED((_NPAD, _LANES), jnp.float32),
            pltpu.SemaphoreType.DMA,
            pltpu.SemaphoreType.DMA,
        ],
    )
    def k(dst_hbm, out_hbm, eb0, eb1, ones, zbuf, acc, s0, s1):
        c = lax.axis_index("c")
        s = lax.axis_index("s")
        wid = c * _NS + s
        @pl.loop(0, kc)
        def _(r):
            ones.at[pl.ds(r, 1), pl.ds(0, _LANES)][...] = (
                jnp.ones((1, _LANES), jnp.float32))
        _zero_rows(zbuf, 128, _LANES)
        base_row = s * rows_per_sub
        _fill_spmem(zbuf, 128, acc, base_row, rows_per_sub)
        plsc.subcore_barrier()
        base_e = wid * per_w

        def dst_slice(chunk):
            return dst_hbm.at[pl.ds(base_e + chunk * kc, kc)]

        pltpu.async_copy(dst_slice(0), eb0, s0)

        @pl.loop(0, n_pairs)
        def _(p):
            a = p * 2
            pltpu.async_copy(dst_slice(a + 1), eb1, s1)
            pltpu.make_async_copy(dst_slice(a), eb0, s0).wait()
            pltpu.sync_copy(ones, acc.at[eb0], add=True)
            @pl.when(p < n_pairs - 1)
            def _():
                pltpu.async_copy(dst_slice(a + 2), eb0, s0)
            pltpu.make_async_copy(dst_slice(a + 1), eb1, s1).wait()
            pltpu.sync_copy(ones, acc.at[eb1], add=True)

        plsc.subcore_barrier()
        pltpu.sync_copy(acc.at[pl.ds(base_row, rows_per_sub)],
                        out_hbm.at[c, pl.ds(base_row, rows_per_sub)])

    return k(dst)


_KA = 104
_NBUF = 3


def _sc_agg(hp, src, dst):
    n, d = hp.shape
    e = src.shape[0]
    per_w = e // _NW
    n_full = per_w // _KA
    tail = per_w - n_full * _KA
    n_q = n_full // _NBUF
    assert n_full == n_q * _NBUF
    rows_per_sub = _NPAD // _NS
    mesh = plsc.VectorSubcoreMesh(core_axis_name="c", subcore_axis_name="s")

    @functools.partial(
        pl.kernel,
        out_type=jax.ShapeDtypeStruct((_NC, _NPAD, d), jnp.float32),
        mesh=mesh,
        scratch_types=(
            [pltpu.VMEM((_KA,), jnp.int32)] * (2 * _NBUF)
            + [pltpu.VMEM((max(tail, 8),), jnp.int32)] * 2
            + [pltpu.VMEM((_KA, d), jnp.float32)] * _NBUF
            + [pltpu.VMEM_SHARED((_NPAD, d), jnp.float32)]
            + [pltpu.SemaphoreType.DMA] * (3 * _NBUF)
        ),
    )
    def k(hp_hbm, src_hbm, dst_hbm, out_hbm, *bufs):
        sb = bufs[0:_NBUF]
        db = bufs[_NBUF:2 * _NBUF]
        sbt, dbt = bufs[2 * _NBUF:2 * _NBUF + 2]
        rows = bufs[2 * _NBUF + 2:3 * _NBUF + 2]
        acc = bufs[3 * _NBUF + 2]
        g = bufs[3 * _NBUF + 3:4 * _NBUF + 3]
        es = bufs[4 * _NBUF + 3:5 * _NBUF + 3]
        ed = bufs[5 * _NBUF + 3:6 * _NBUF + 3]
        c = lax.axis_index("c")
        s = lax.axis_index("s")
        wid = c * _NS + s
        _zero_rows(rows[0], min(_KA, 128), d)
        base_row = s * rows_per_sub
        _fill_spmem(rows[0], min(_KA, 128), acc, base_row, rows_per_sub)
        plsc.subcore_barrier()
        base_e = wid * per_w

        def sidx(chunk, size=_KA):
            return src_hbm.at[pl.ds(base_e + chunk * _KA, size)]

        def didx(chunk, size=_KA):
            return dst_hbm.at[pl.ds(base_e + chunk * _KA, size)]

        for j in range(_NBUF):
            pltpu.async_copy(sidx(j), sb[j], es[j])
            pltpu.async_copy(didx(j), db[j], ed[j])
        for j in range(_NBUF):
            pltpu.make_async_copy(sidx(j), sb[j], es[j]).wait()
            pltpu.async_copy(hp_hbm.at[sb[j]], rows[j], g[j])

        @pl.loop(0, n_q)
        def _(q):
            base_c = q * _NBUF
            more = q < n_q - 1
            for j in range(_NBUF):
                ch = base_c + j
                pltpu.make_async_copy(hp_hbm.at[sb[j]], rows[j], g[j]).wait()
                @pl.when(more)
                def _(j=j, ch=ch):
                    pltpu.async_copy(sidx(ch + _NBUF), sb[j], es[j])
                pltpu.make_async_copy(didx(ch), db[j], ed[j]).wait()
                pltpu.sync_copy(rows[j], acc.at[db[j]], add=True)
                @pl.when(more)
                def _(j=j, ch=ch):
                    pltpu.async_copy(didx(ch + _NBUF), db[j], ed[j])
                    pltpu.make_async_copy(sidx(ch + _NBUF), sb[j],
                                          es[j]).wait()
                    pltpu.async_copy(hp_hbm.at[sb[j]], rows[j], g[j])

        if tail:
            pltpu.sync_copy(sidx(n_full, tail), sbt)
            pltpu.sync_copy(didx(n_full, tail), dbt)
            pltpu.sync_copy(hp_hbm.at[sbt], rows[0].at[pl.ds(0, tail)])
            pltpu.sync_copy(rows[0].at[pl.ds(0, tail)], acc.at[dbt],
                            add=True)

        plsc.subcore_barrier()
        pltpu.sync_copy(acc.at[pl.ds(base_row, rows_per_sub)],
                        out_hbm.at[c, pl.ds(base_row, rows_per_sub)])

    return k(hp, src, dst)


_BLK = 1000


def _tc_stage1(x, W1, degp):
    n, d = x.shape
    h = W1.shape[1]
    grid = (n // _BLK,)

    def body(x_ref, w_ref, degp_ref, hp_ref, dinv_ref):
        deg = 1.0 + degp_ref[0, :, 0:1] + degp_ref[1, :, 0:1]
        dinv = lax.rsqrt(deg)
        acc = jnp.dot(x_ref[...], w_ref[...],
                      preferred_element_type=jnp.float32)
        hp_ref[...] = acc * dinv
        dinv_ref[...] = dinv

    return pl.pallas_call(
        body,
        grid=grid,
        in_specs=[
            pl.BlockSpec((_BLK, d), lambda i: (i, 0)),
            pl.BlockSpec((d, h), lambda i: (0, 0)),
            pl.BlockSpec((_NC, _BLK, _LANES), lambda i: (0, i, 0)),
        ],
        out_specs=[
            pl.BlockSpec((_BLK, h), lambda i: (i, 0)),
            pl.BlockSpec((_BLK, 1), lambda i: (i, 0)),
        ],
        out_shape=[
            jax.ShapeDtypeStruct((n, h), jnp.float32),
            jax.ShapeDtypeStruct((n, 1), jnp.float32),
        ],
    )(x, W1, degp)


def _tc_stage2(aggp, hp1, dinv, b1, W2):
    n, h = hp1.shape
    grid = (n // _BLK,)

    def body(aggp_ref, hp_ref, dinv_ref, b_ref, w_ref, out1_ref, hp2_ref):
        agg = aggp_ref[0] + aggp_ref[1]
        o1 = jnp.maximum(
            dinv_ref[...] * (agg + hp_ref[...]) + b_ref[...], 0.0)
        out1_ref[...] = o1
        hp2_ref[...] = dinv_ref[...] * jnp.dot(
            o1, w_ref[...], preferred_element_type=jnp.float32)

    return pl.pallas_call(
        body,
        grid=grid,
        in_specs=[
            pl.BlockSpec((_NC, _BLK, h), lambda i: (0, i, 0)),
            pl.BlockSpec((_BLK, h), lambda i: (i, 0)),
            pl.BlockSpec((_BLK, 1), lambda i: (i, 0)),
            pl.BlockSpec((1, h), lambda i: (0, 0)),
            pl.BlockSpec((h, h), lambda i: (0, 0)),
        ],
        out_specs=[
            pl.BlockSpec((_BLK, h), lambda i: (i, 0)),
            pl.BlockSpec((_BLK, h), lambda i: (i, 0)),
        ],
        out_shape=[
            jax.ShapeDtypeStruct((n, h), jnp.float32),
            jax.ShapeDtypeStruct((n, h), jnp.float32),
        ],
    )(aggp, hp1, dinv, b1[None, :], W2)


def _tc_stage3(aggp, hp2, dinv, b2, out1, Wc, bc):
    n, h = hp2.shape
    o = Wc.shape[1]
    grid = (n // _BLK,)

    def body(aggp_ref, hp_ref, dinv_ref, b_ref, out1_ref, wc_ref, bc_ref,
             out_ref):
        agg = aggp_ref[0] + aggp_ref[1]
        o2 = jnp.maximum(
            dinv_ref[...] * (agg + hp_ref[...]) + b_ref[...], 0.0)
        out_ref[...] = jnp.dot(o2 + out1_ref[...], wc_ref[...],
                               preferred_element_type=jnp.float32) + bc_ref[...]

    return pl.pallas_call(
        body,
        grid=grid,
        in_specs=[
            pl.BlockSpec((_NC, _BLK, h), lambda i: (0, i, 0)),
            pl.BlockSpec((_BLK, h), lambda i: (i, 0)),
            pl.BlockSpec((_BLK, 1), lambda i: (i, 0)),
            pl.BlockSpec((1, h), lambda i: (0, 0)),
            pl.BlockSpec((_BLK, h), lambda i: (i, 0)),
            pl.BlockSpec((h, o), lambda i: (0, 0)),
            pl.BlockSpec((1, o), lambda i: (0, 0)),
        ],
        out_specs=pl.BlockSpec((_BLK, o), lambda i: (i, 0)),
        out_shape=jax.ShapeDtypeStruct((n, o), jnp.float32),
    )(aggp, hp2, dinv, b2[None, :], out1, Wc, bc[None, :])


def kernel(x, edge_index, W1, b1, W2, b2, Wc, bc):
    src = edge_index[0]
    dst = edge_index[1]
    degp = _sc_degree(dst)
    hp1, dinv = _tc_stage1(x, W1, degp)
    aggp1 = _sc_agg(hp1, src, dst)
    out1, hp2 = _tc_stage2(aggp1, hp1, dinv, b1, W2)
    aggp2 = _sc_agg(hp2, src, dst)
    return _tc_stage3(aggp2, hp2, dinv, b2, out1, Wc, bc)
